# baseline (speedup 1.0000x reference)
import functools

import jax
import jax.numpy as jnp
from jax import lax
from jax.experimental import pallas as pl
from jax.experimental.pallas import tpu as pltpu
from jax.experimental.pallas import tpu_sc as plsc

B = 16384
D = 16

_info = plsc.get_sparse_core_info()
_NC, _NS = _info.num_cores, _info.num_subcores
NW = _NC * _NS
BPW = B // NW
CHUNK = 128
NCHUNK = BPW // CHUNK

_sc_mesh = plsc.VectorSubcoreMesh(core_axis_name="c", subcore_axis_name="s")


@functools.partial(
    pl.kernel,
    mesh=_sc_mesh,
    out_type=[
        jax.ShapeDtypeStruct((B, D), jnp.float32),
        jax.ShapeDtypeStruct((B, D), jnp.float32),
    ],
    scratch_types=[
        pltpu.VMEM((NCHUNK, CHUNK), jnp.int32),
        pltpu.VMEM((BPW, D), jnp.float32),
        pltpu.VMEM((NCHUNK, CHUNK), jnp.int32),
        pltpu.VMEM((BPW, D), jnp.float32),
        pltpu.SemaphoreType.DMA,
        pltpu.SemaphoreType.DMA,
    ],
    compiler_params=pltpu.CompilerParams(use_tc_tiling_on_sc=False),
)
def _sc_gather(uid_hbm, mid_hbm, uid_tab, mid_tab, euid_out, emid_out,
               uidx_v, urows_v, midx_v, mrows_v, usem, msem):
    wid = lax.axis_index("s") * _NC + lax.axis_index("c")
    base = wid * BPW
    pltpu.sync_copy(uid_hbm.at[wid], uidx_v)
    pltpu.sync_copy(mid_hbm.at[wid], midx_v)
    ucopies = []
    mcopies = []
    for ci in range(NCHUNK):
        ucopies.append(pltpu.async_copy(
            uid_tab.at[uidx_v.at[ci]],
            urows_v.at[pl.ds(ci * CHUNK, CHUNK)], usem))
        mcopies.append(pltpu.async_copy(
            mid_tab.at[midx_v.at[ci]],
            mrows_v.at[pl.ds(ci * CHUNK, CHUNK)], msem))
    for c in ucopies:
        c.wait()
    for c in mcopies:
        c.wait()
    pltpu.sync_copy(urows_v, euid_out.at[pl.ds(base, BPW)])
    pltpu.sync_copy(mrows_v, emid_out.at[pl.ds(base, BPW)])


BLK = 2048
GRID = B // BLK


def _tc_body(gidx_ref, aidx_ref, jidx_ref, euid_ref, emid_ref,
             wcat_ref, tab_ref, aux_ref, out_ref, lam_ref):
    euid = euid_ref[...]
    emid = emid_ref[...]
    g = gidx_ref[0, 0, :]
    a = aidx_ref[0, 0, :]
    j = jidx_ref[0, 0, :]
    tab = tab_ref[...]
    w1a = wcat_ref[:D, :]
    w1b = wcat_ref[D:, :]
    b1row = aux_ref[0:1, :]
    w2row = aux_ref[1:2, :]
    wsvda = aux_ref[2:3, :]
    wsvdb = aux_ref[3:4, :]
    bsvd = aux_ref[4, 0]

    iota = lax.broadcasted_iota(jnp.int32, (BLK, 32), 1)
    oh_g = (g[:, None] == iota).astype(jnp.float32)
    oh_a = ((a[:, None] + 2) == iota).astype(jnp.float32)
    oh_j = ((j[:, None] + 9) == iota).astype(jnp.float32)
    eg = jnp.dot(oh_g, tab, preferred_element_type=jnp.float32)
    ea = jnp.dot(oh_a, tab, preferred_element_type=jnp.float32)
    ej = jnp.dot(oh_j, tab, preferred_element_type=jnp.float32)

    m1 = jnp.dot(emid, w1a, preferred_element_type=jnp.float32) + b1row

    def score(e):
        h = jnp.tanh(m1 + jnp.dot(e, w1b, preferred_element_type=jnp.float32))
        return jnp.sum(h * w2row, axis=1, keepdims=True)

    s1, s2, s3 = score(eg), score(ea), score(ej)
    mx = jnp.maximum(jnp.maximum(s1, s2), s3)
    x1 = jnp.exp(s1 - mx)
    x2 = jnp.exp(s2 - mx)
    x3 = jnp.exp(s3 - mx)
    den = x1 + x2 + x3
    l1, l2, l3 = x1 / den, x2 / den, x3 / den

    fu = l1 * eg + l2 * ea + l3 * ej + euid
    out_ref[...] = jnp.sum(fu * wsvda + emid * wsvdb, axis=1, keepdims=True) + bsvd
    lam_ref[...] = jnp.concatenate([l1, l2, l3], axis=1)


def kernel(uid_table, gender_table, age_table, job_table, mid_table,
           W1, b1, W2, b2, W_svd, b_svd,
           uid, gender, age, job, mid):
    uid = uid.astype(jnp.int32).reshape(NW, NCHUNK, CHUNK)
    mid = mid.astype(jnp.int32).reshape(NW, NCHUNK, CHUNK)
    euid, emid = _sc_gather(uid, mid, uid_table, mid_table)

    tab = jnp.zeros((32, D), jnp.float32)
    tab = tab.at[0:2].set(gender_table).at[2:9].set(age_table).at[9:30].set(job_table)
    aux = jnp.zeros((8, D), jnp.float32)
    aux = (aux.at[0].set(b1)
              .at[1].set(W2[:, 0])
              .at[2].set(W_svd[:D, 0])
              .at[3].set(W_svd[D:, 0])
              .at[4, 0].set(b_svd[0]))

    g3 = gender.astype(jnp.int32).reshape(GRID, 1, BLK)
    a3 = age.astype(jnp.int32).reshape(GRID, 1, BLK)
    j3 = job.astype(jnp.int32).reshape(GRID, 1, BLK)

    idx_spec = pl.BlockSpec((1, 1, BLK), lambda i: (i, 0, 0))
    row_spec = pl.BlockSpec((BLK, D), lambda i: (i, 0))
    full = lambda shape: pl.BlockSpec(shape, lambda i: (0, 0))

    out, lam = pl.pallas_call(
        _tc_body,
        grid=(GRID,),
        in_specs=[idx_spec, idx_spec, idx_spec, row_spec, row_spec,
                  full((2 * D, D)), full((32, D)), full((8, D))],
        out_specs=[pl.BlockSpec((BLK, 1), lambda i: (i, 0)),
                   pl.BlockSpec((BLK, 3), lambda i: (i, 0))],
        out_shape=[jax.ShapeDtypeStruct((B, 1), jnp.float32),
                   jax.ShapeDtypeStruct((B, 3), jnp.float32)],
    )(g3, a3, j3, euid, emid, W1, tab, aux)

    return (out, lam.reshape(B, 3, 1))
